# Initial kernel scaffold; baseline (speedup 1.0000x reference)
import jax
import jax.numpy as jnp
from jax.experimental import pallas as pl


def kernel(ipos, tables, W1, b1, W2, b2, W3, b3):
    raise NotImplementedError("write your pallas kernel here")


# speedup vs baseline: 75.5886x; 75.5886x over previous
import functools
import math

import jax
import jax.numpy as jnp
import numpy as np
from jax import lax
from jax.experimental import pallas as pl
from jax.experimental.pallas import tpu as pltpu
from jax.experimental.pallas import tpu_sc as plsc

N_LEVELS = 16
N_FEATS = 4
LOG2_HASH = 18
BASE_RES = 16
FINEST_RES = 512
N_POINTS = 262144

_b = math.exp((math.log(FINEST_RES) - math.log(BASE_RES)) / (N_LEVELS - 1))
_RES = [int(math.floor(BASE_RES * (_b ** l))) for l in range(N_LEVELS)]
_SIZES = [min(r ** 3, 2 ** LOG2_HASH) for r in _RES]
_P2 = np.int32(np.uint32(2654435761).view(np.int32))
_P3 = np.int32(np.uint32(805459861).view(np.int32))

NW = 32
LANES = 16
CHUNK = 512
PW = N_POINTS // NW
CPB = CHUNK // 128
NB = 8 * CPB


def _is_pow2(m):
    return (m & (m - 1)) == 0


def _umod(h, m):
    hf = h.astype(jnp.float32)
    hf = jnp.where(h < 0, hf + jnp.float32(4294967296.0), hf)
    q = (hf * jnp.float32(1.0 / m)).astype(jnp.int32)
    r = h - q * jnp.int32(m)
    r = jnp.where(r < 0, r + jnp.int32(m), r)
    r = jnp.where(r >= jnp.int32(m), r - jnp.int32(m), r)
    return r


def _encode_body(x_hbm, y_hbm, z_hbm, *rest):
    tables = rest[:N_LEVELS]
    out_hbm = rest[N_LEVELS]
    (xb, yb, zb, xfx, xfy, xfz, idx_buf, g_buf, feats_buf, sem) = rest[N_LEVELS + 1:]

    wid = lax.axis_index("s") * 2 + lax.axis_index("c")
    base = wid * PW
    iota = lax.iota(jnp.int32, LANES)
    iota64 = iota * 64
    fvecs = [jnp.full((LANES,), f, jnp.int32) for f in range(4)]

    def chunk_body(k, _):
        start = base + k * CHUNK
        pltpu.sync_copy(x_hbm.at[pl.ds(start, CHUNK)], xb)
        pltpu.sync_copy(y_hbm.at[pl.ds(start, CHUNK)], yb)
        pltpu.sync_copy(z_hbm.at[pl.ds(start, CHUNK)], zb)

        for l in range(N_LEVELS):
            r = _RES[l]
            m = _SIZES[l]

            def hash_body(i, _, r=r, m=m):
                xs = xb[pl.ds(i * LANES, LANES)] * jnp.float32(r)
                ys = yb[pl.ds(i * LANES, LANES)] * jnp.float32(r)
                zs = zb[pl.ds(i * LANES, LANES)] * jnp.float32(r)
                xi = xs.astype(jnp.int32)
                yi = ys.astype(jnp.int32)
                zi = zs.astype(jnp.int32)
                xfx[pl.ds(i * LANES, LANES)] = xs - xi.astype(jnp.float32)
                xfy[pl.ds(i * LANES, LANES)] = ys - yi.astype(jnp.float32)
                xfz[pl.ds(i * LANES, LANES)] = zs - zi.astype(jnp.float32)
                hx0, hx1 = xi, xi + 1
                hy0 = yi * _P2
                hy1 = hy0 + _P2
                hz0 = zi * _P3
                hz1 = hz0 + _P3
                r0 = i * LANES - (i // 8) * 128
                for c in range(8):
                    h = (hx1 if (c >> 2) & 1 else hx0) ^ \
                        (hy1 if (c >> 1) & 1 else hy0) ^ \
                        (hz1 if c & 1 else hz0)
                    if _is_pow2(m):
                        idx = h & jnp.int32(m - 1)
                    else:
                        idx = _umod(h, m)
                    idx_buf[c * CPB + i // 8, pl.ds(r0, LANES)] = idx
                return 0

            lax.fori_loop(0, CHUNK // LANES, hash_body, 0)

            tbl = tables[l]

            def fire(b, _):
                pltpu.make_async_copy(tbl.at[idx_buf.at[b]],
                                      g_buf.at[pl.ds(b * 128, 128)], sem).start()
                return 0

            lax.fori_loop(0, NB, fire, 0)

            def drain(b, _):
                pltpu.make_async_copy(tbl.at[idx_buf.at[b]],
                                      g_buf.at[pl.ds(b * 128, 128)], sem).wait()
                return 0

            lax.fori_loop(0, NB, drain, 0)

            def accum_body(i, _, l=l):
                tx1 = xfx[pl.ds(i * LANES, LANES)]
                ty1 = xfy[pl.ds(i * LANES, LANES)]
                tz1 = xfz[pl.ds(i * LANES, LANES)]
                tx0 = 1.0 - tx1
                ty0 = 1.0 - ty1
                tz0 = 1.0 - tz1
                rv0 = iota + i * LANES
                acc = [jnp.zeros((LANES,), jnp.float32) for _ in range(4)]
                for c in range(8):
                    w = ((tx1 if (c >> 2) & 1 else tx0)
                         * (ty1 if (c >> 1) & 1 else ty0)
                         * (tz1 if c & 1 else tz0))
                    rv = rv0 + c * CHUNK
                    for f in range(4):
                        g = plsc.load_gather(g_buf, [rv, fvecs[f]])
                        acc[f] = acc[f] + g * w
                off = i * (LANES * 64) + l * 4
                for f in range(4):
                    plsc.store_scatter(feats_buf, [iota64 + (off + f)], acc[f])
                return 0

            lax.fori_loop(0, CHUNK // LANES, accum_body, 0)

        pltpu.sync_copy(feats_buf, out_hbm.at[pl.ds(start * 64, CHUNK * 64)])
        return 0

    lax.fori_loop(0, PW // CHUNK, chunk_body, 0)


@jax.jit
def _encode(x, y, z, *tables):
    fn = pl.kernel(
        _encode_body,
        out_type=jax.ShapeDtypeStruct((N_POINTS * 64,), jnp.float32),
        mesh=plsc.VectorSubcoreMesh(core_axis_name="c", subcore_axis_name="s"),
        compiler_params=pltpu.CompilerParams(
            needs_layout_passes=False, use_tc_tiling_on_sc=False
        ),
        scratch_types=[
            pltpu.VMEM((CHUNK,), jnp.float32),
            pltpu.VMEM((CHUNK,), jnp.float32),
            pltpu.VMEM((CHUNK,), jnp.float32),
            pltpu.VMEM((CHUNK,), jnp.float32),
            pltpu.VMEM((CHUNK,), jnp.float32),
            pltpu.VMEM((CHUNK,), jnp.float32),
            pltpu.VMEM((NB, 128), jnp.int32),
            pltpu.VMEM((NB * 128, 4), jnp.float32),
            pltpu.VMEM((CHUNK * 64,), jnp.float32),
            pltpu.SemaphoreType.DMA,
        ],
    )
    return fn(x, y, z, *tables)


BM = 2048


def _mlp_body(f_ref, w1_ref, b1_ref, w2_ref, b2_ref, w3_ref, b3_ref, o_ref):
    x = f_ref[...]
    h = jnp.dot(x, w1_ref[...], preferred_element_type=jnp.float32) + b1_ref[...]
    h = jnp.maximum(h, 0.0)
    h = jnp.dot(h, w2_ref[...], preferred_element_type=jnp.float32) + b2_ref[...]
    h = jnp.maximum(h, 0.0)
    t = jnp.dot(h, w3_ref[...], preferred_element_type=jnp.float32) + b3_ref[...]
    o_ref[...] = 1.0 / (1.0 + jnp.exp(-t))


@functools.partial(jax.jit, static_argnames=("n_out",))
def _mlp(feats, W1, b1, W2, b2, W3, b3, n_out):
    return pl.pallas_call(
        _mlp_body,
        grid=(N_POINTS // BM,),
        in_specs=[
            pl.BlockSpec((BM, 64), lambda i: (i, 0)),
            pl.BlockSpec((64, 64), lambda i: (0, 0)),
            pl.BlockSpec((1, 64), lambda i: (0, 0)),
            pl.BlockSpec((64, 64), lambda i: (0, 0)),
            pl.BlockSpec((1, 64), lambda i: (0, 0)),
            pl.BlockSpec((64, n_out), lambda i: (0, 0)),
            pl.BlockSpec((1, n_out), lambda i: (0, 0)),
        ],
        out_specs=pl.BlockSpec((BM, n_out), lambda i: (i, 0)),
        out_shape=jax.ShapeDtypeStruct((N_POINTS, n_out), jnp.float32),
    )(feats, W1, b1, W2, b2, W3, b3)


def kernel(ipos, tables, W1, b1, W2, b2, W3, b3):
    x = ipos[:, 0]
    y = ipos[:, 1]
    z = ipos[:, 2]
    feats = _encode(x, y, z, *tables).reshape(N_POINTS, 64)
    n_out = W3.shape[1]
    return _mlp(feats, W1, b1[None, :], W2, b2[None, :], W3, b3[None, :], n_out)


# speedup vs baseline: 96.0609x; 1.2708x over previous
import functools
import math

import jax
import jax.numpy as jnp
import numpy as np
from jax import lax
from jax.experimental import pallas as pl
from jax.experimental.pallas import tpu as pltpu
from jax.experimental.pallas import tpu_sc as plsc

N_LEVELS = 16
N_FEATS = 4
LOG2_HASH = 18
BASE_RES = 16
FINEST_RES = 512
N_POINTS = 262144

_b = math.exp((math.log(FINEST_RES) - math.log(BASE_RES)) / (N_LEVELS - 1))
_RES = [int(math.floor(BASE_RES * (_b ** l))) for l in range(N_LEVELS)]
_SIZES = [min(r ** 3, 2 ** LOG2_HASH) for r in _RES]
_P2 = np.int32(np.uint32(2654435761).view(np.int32))
_P3 = np.int32(np.uint32(805459861).view(np.int32))

NW = 32
LANES = 16
CHUNK = 512
PW = N_POINTS // NW
CPB = CHUNK // 128
NB = 8 * CPB


def _is_pow2(m):
    return (m & (m - 1)) == 0


def _umod(h, m):
    hf = h.astype(jnp.float32)
    hf = jnp.where(h < 0, hf + jnp.float32(4294967296.0), hf)
    q = (hf * jnp.float32(1.0 / m)).astype(jnp.int32)
    r = h - q * jnp.int32(m)
    r = jnp.where(r < 0, r + jnp.int32(m), r)
    r = jnp.where(r >= jnp.int32(m), r - jnp.int32(m), r)
    return r


_BASES = [0]
for _s in _SIZES:
    _BASES.append(_BASES[-1] + _s)
TOTAL_ROWS = _BASES[N_LEVELS]


def _encode_body(x_hbm, y_hbm, z_hbm, tbl, out_hbm,
                 xb, yb, zb, xfx, xfy, xfz, idx_buf, g_buf, feats_buf, sem):

    wid = lax.axis_index("s") * 2 + lax.axis_index("c")
    base = wid * PW
    iota = lax.iota(jnp.int32, LANES)
    iota64 = iota * 64
    fvecs = [jnp.full((LANES,), f, jnp.int32) for f in range(4)]

    def chunk_body(k, _):
        start = base + k * CHUNK
        pltpu.sync_copy(x_hbm.at[pl.ds(start, CHUNK)], xb)
        pltpu.sync_copy(y_hbm.at[pl.ds(start, CHUNK)], yb)
        pltpu.sync_copy(z_hbm.at[pl.ds(start, CHUNK)], zb)

        for l in range(N_LEVELS):
            r = _RES[l]
            m = _SIZES[l]

            bs = _BASES[l]

            def hash_body(i, _, r=r, m=m, bs=bs):
                xs = xb[pl.ds(i * LANES, LANES)] * jnp.float32(r)
                ys = yb[pl.ds(i * LANES, LANES)] * jnp.float32(r)
                zs = zb[pl.ds(i * LANES, LANES)] * jnp.float32(r)
                xi = xs.astype(jnp.int32)
                yi = ys.astype(jnp.int32)
                zi = zs.astype(jnp.int32)
                xfx[pl.ds(i * LANES, LANES)] = xs - xi.astype(jnp.float32)
                xfy[pl.ds(i * LANES, LANES)] = ys - yi.astype(jnp.float32)
                xfz[pl.ds(i * LANES, LANES)] = zs - zi.astype(jnp.float32)
                hx0, hx1 = xi, xi + 1
                hy0 = yi * _P2
                hy1 = hy0 + _P2
                hz0 = zi * _P3
                hz1 = hz0 + _P3
                r0 = i * LANES - (i // 8) * 128
                for c in range(8):
                    h = (hx1 if (c >> 2) & 1 else hx0) ^ \
                        (hy1 if (c >> 1) & 1 else hy0) ^ \
                        (hz1 if c & 1 else hz0)
                    if _is_pow2(m):
                        idx = h & jnp.int32(m - 1)
                    else:
                        idx = _umod(h, m)
                    idx_buf[c * CPB + i // 8, pl.ds(r0, LANES)] = idx + jnp.int32(bs)
                return 0

            lax.fori_loop(0, CHUNK // LANES, hash_body, 0)

            def fire(b, _):
                pltpu.make_async_copy(tbl.at[idx_buf.at[b]],
                                      g_buf.at[pl.ds(b * 128, 128)], sem).start()
                return 0

            lax.fori_loop(0, NB, fire, 0)

            def drain(b, _):
                pltpu.make_async_copy(tbl.at[idx_buf.at[b]],
                                      g_buf.at[pl.ds(b * 128, 128)], sem).wait()
                return 0

            lax.fori_loop(0, NB, drain, 0)

            def accum_body(i, _, l=l):
                tx1 = xfx[pl.ds(i * LANES, LANES)]
                ty1 = xfy[pl.ds(i * LANES, LANES)]
                tz1 = xfz[pl.ds(i * LANES, LANES)]
                tx0 = 1.0 - tx1
                ty0 = 1.0 - ty1
                tz0 = 1.0 - tz1
                rv0 = iota + i * LANES
                acc = [jnp.zeros((LANES,), jnp.float32) for _ in range(4)]
                for c in range(8):
                    w = ((tx1 if (c >> 2) & 1 else tx0)
                         * (ty1 if (c >> 1) & 1 else ty0)
                         * (tz1 if c & 1 else tz0))
                    rv = rv0 + c * CHUNK
                    for f in range(4):
                        g = plsc.load_gather(g_buf, [rv, fvecs[f]])
                        acc[f] = acc[f] + g * w
                off = i * (LANES * 64) + l * 4
                for f in range(4):
                    plsc.store_scatter(feats_buf, [iota64 + (off + f)], acc[f])
                return 0

            lax.fori_loop(0, CHUNK // LANES, accum_body, 0)

        pltpu.sync_copy(feats_buf, out_hbm.at[pl.ds(start * 64, CHUNK * 64)])
        return 0

    lax.fori_loop(0, PW // CHUNK, chunk_body, 0)


@jax.jit
def _encode(x, y, z, tbl):
    fn = pl.kernel(
        _encode_body,
        out_type=jax.ShapeDtypeStruct((N_POINTS * 64,), jnp.float32),
        mesh=plsc.VectorSubcoreMesh(core_axis_name="c", subcore_axis_name="s"),
        compiler_params=pltpu.CompilerParams(
            needs_layout_passes=False, use_tc_tiling_on_sc=False
        ),
        scratch_types=[
            pltpu.VMEM((CHUNK,), jnp.float32),
            pltpu.VMEM((CHUNK,), jnp.float32),
            pltpu.VMEM((CHUNK,), jnp.float32),
            pltpu.VMEM((CHUNK,), jnp.float32),
            pltpu.VMEM((CHUNK,), jnp.float32),
            pltpu.VMEM((CHUNK,), jnp.float32),
            pltpu.VMEM((NB, 128), jnp.int32),
            pltpu.VMEM((NB * 128, 8), jnp.float32),
            pltpu.VMEM((CHUNK * 64,), jnp.float32),
            pltpu.SemaphoreType.DMA,
        ],
    )
    return fn(x, y, z, tbl)


BM = 2048


def _mlp_body(f_ref, w1_ref, b1_ref, w2_ref, b2_ref, w3_ref, b3_ref, o_ref):
    x = f_ref[...]
    h = jnp.dot(x, w1_ref[...], preferred_element_type=jnp.float32) + b1_ref[...]
    h = jnp.maximum(h, 0.0)
    h = jnp.dot(h, w2_ref[...], preferred_element_type=jnp.float32) + b2_ref[...]
    h = jnp.maximum(h, 0.0)
    t = jnp.dot(h, w3_ref[...], preferred_element_type=jnp.float32) + b3_ref[...]
    o_ref[...] = 1.0 / (1.0 + jnp.exp(-t))


@functools.partial(jax.jit, static_argnames=("n_out",))
def _mlp(feats, W1, b1, W2, b2, W3, b3, n_out):
    return pl.pallas_call(
        _mlp_body,
        grid=(N_POINTS // BM,),
        in_specs=[
            pl.BlockSpec((BM, 64), lambda i: (i, 0)),
            pl.BlockSpec((64, 64), lambda i: (0, 0)),
            pl.BlockSpec((1, 64), lambda i: (0, 0)),
            pl.BlockSpec((64, 64), lambda i: (0, 0)),
            pl.BlockSpec((1, 64), lambda i: (0, 0)),
            pl.BlockSpec((64, n_out), lambda i: (0, 0)),
            pl.BlockSpec((1, n_out), lambda i: (0, 0)),
        ],
        out_specs=pl.BlockSpec((BM, n_out), lambda i: (i, 0)),
        out_shape=jax.ShapeDtypeStruct((N_POINTS, n_out), jnp.float32),
    )(feats, W1, b1, W2, b2, W3, b3)


def kernel(ipos, tables, W1, b1, W2, b2, W3, b3):
    x = ipos[:, 0]
    y = ipos[:, 1]
    z = ipos[:, 2]
    tbl = jnp.concatenate([jnp.pad(t, ((0, 0), (0, 4))) for t in tables], axis=0)
    feats = _encode(x, y, z, tbl).reshape(N_POINTS, 64)
    n_out = W3.shape[1]
    return _mlp(feats, W1, b1[None, :], W2, b2[None, :], W3, b3[None, :], n_out)


# speedup vs baseline: 123.2014x; 1.2825x over previous
import functools
import math

import jax
import jax.numpy as jnp
import numpy as np
from jax import lax
from jax.experimental import pallas as pl
from jax.experimental.pallas import tpu as pltpu
from jax.experimental.pallas import tpu_sc as plsc

N_LEVELS = 16
N_FEATS = 4
LOG2_HASH = 18
BASE_RES = 16
FINEST_RES = 512
N_POINTS = 262144

_b = math.exp((math.log(FINEST_RES) - math.log(BASE_RES)) / (N_LEVELS - 1))
_RES = [int(math.floor(BASE_RES * (_b ** l))) for l in range(N_LEVELS)]
_SIZES = [min(r ** 3, 2 ** LOG2_HASH) for r in _RES]
_P2 = np.int32(np.uint32(2654435761).view(np.int32))
_P3 = np.int32(np.uint32(805459861).view(np.int32))

NW = 32
LANES = 16
CHUNK = 512
PW = N_POINTS // NW
CPB = CHUNK // 128
NB = 8 * CPB


def _is_pow2(m):
    return (m & (m - 1)) == 0


def _umod(h, m):
    hf = h.astype(jnp.float32)
    hf = jnp.where(h < 0, hf + jnp.float32(4294967296.0), hf)
    q = (hf * jnp.float32(1.0 / m)).astype(jnp.int32)
    r = h - q * jnp.int32(m)
    r = jnp.where(r < 0, r + jnp.int32(m), r)
    r = jnp.where(r >= jnp.int32(m), r - jnp.int32(m), r)
    return r


_BASES = [0]
for _s in _SIZES:
    _BASES.append(_BASES[-1] + _s)
TOTAL_ROWS = _BASES[N_LEVELS]

_CH = [-(-(-(-s // -NW)) // -LANES) * LANES for s in _SIZES]
_CH = [((s + NW - 1) // NW + LANES - 1) // LANES * LANES for s in _SIZES]
_BP = [0]
for _c in _CH:
    _BP.append(_BP[-1] + NW * _c)
RP = _BP[N_LEVELS]
_PL = [(s + 7) // 8 * 8 for s in _SIZES]
_SEG = [0]
for _p in _PL:
    _SEG.append(_SEG[-1] + 4 * _p)


def _relayout_body(src_hbm, big_hbm, p0, p1, p2, p3, rowbuf, sem):
    wid = lax.axis_index("s") * 2 + lax.axis_index("c")
    iota = lax.iota(jnp.int32, LANES)
    pat = iota * 8
    planes = (p0, p1, p2, p3)
    for l in range(N_LEVELS):
        ch = _CH[l]
        seg = _SEG[l]
        pln = _PL[l]
        r0 = wid * ch
        for f in range(4):
            pltpu.sync_copy(src_hbm.at[pl.ds(seg + f * pln + r0, ch)],
                            planes[f].at[pl.ds(0, ch)])

        def grp(g, _, ch=ch):
            for f in range(4):
                v = planes[f][pl.ds(g * LANES, LANES)]
                plsc.store_scatter(rowbuf, [pat + (g * 128 + f)], v)
            return 0

        lax.fori_loop(0, ch // LANES, grp, 0)
        pltpu.sync_copy(rowbuf.at[pl.ds(0, ch * 8)],
                        big_hbm.at[pl.ds((_BP[l] + r0) * 8, ch * 8)])


@jax.jit
def _relayout(tall):
    fn = pl.kernel(
        _relayout_body,
        out_type=jax.ShapeDtypeStruct((RP * 8,), jnp.float32),
        mesh=plsc.VectorSubcoreMesh(core_axis_name="c", subcore_axis_name="s"),
        compiler_params=pltpu.CompilerParams(
            needs_layout_passes=False, use_tc_tiling_on_sc=False
        ),
        scratch_types=[
            pltpu.VMEM((max(_CH),), jnp.float32),
            pltpu.VMEM((max(_CH),), jnp.float32),
            pltpu.VMEM((max(_CH),), jnp.float32),
            pltpu.VMEM((max(_CH),), jnp.float32),
            pltpu.VMEM((max(_CH) * 8,), jnp.float32),
            pltpu.SemaphoreType.DMA,
        ],
    )
    return fn(tall)


def _encode_body(x_hbm, y_hbm, z_hbm, tbl, out_hbm,
                 xb, yb, zb, xfx, xfy, xfz, idx_buf, g_buf, feats_buf, sem):

    wid = lax.axis_index("s") * 2 + lax.axis_index("c")
    base = wid * PW
    iota = lax.iota(jnp.int32, LANES)
    iota64 = iota * 64
    fvecs = [jnp.full((LANES,), f, jnp.int32) for f in range(4)]

    def chunk_body(k, _):
        start = base + k * CHUNK
        pltpu.sync_copy(x_hbm.at[pl.ds(start, CHUNK)], xb)
        pltpu.sync_copy(y_hbm.at[pl.ds(start, CHUNK)], yb)
        pltpu.sync_copy(z_hbm.at[pl.ds(start, CHUNK)], zb)

        for l in range(N_LEVELS):
            r = _RES[l]
            m = _SIZES[l]

            bs = _BP[l]

            def hash_body(i, _, r=r, m=m, bs=bs):
                xs = xb[pl.ds(i * LANES, LANES)] * jnp.float32(r)
                ys = yb[pl.ds(i * LANES, LANES)] * jnp.float32(r)
                zs = zb[pl.ds(i * LANES, LANES)] * jnp.float32(r)
                xi = xs.astype(jnp.int32)
                yi = ys.astype(jnp.int32)
                zi = zs.astype(jnp.int32)
                xfx[pl.ds(i * LANES, LANES)] = xs - xi.astype(jnp.float32)
                xfy[pl.ds(i * LANES, LANES)] = ys - yi.astype(jnp.float32)
                xfz[pl.ds(i * LANES, LANES)] = zs - zi.astype(jnp.float32)
                hx0, hx1 = xi, xi + 1
                hy0 = yi * _P2
                hy1 = hy0 + _P2
                hz0 = zi * _P3
                hz1 = hz0 + _P3
                r0 = i * LANES - (i // 8) * 128
                for c in range(8):
                    h = (hx1 if (c >> 2) & 1 else hx0) ^ \
                        (hy1 if (c >> 1) & 1 else hy0) ^ \
                        (hz1 if c & 1 else hz0)
                    if _is_pow2(m):
                        idx = h & jnp.int32(m - 1)
                    else:
                        idx = _umod(h, m)
                    idx_buf[c * CPB + i // 8, pl.ds(r0, LANES)] = idx + jnp.int32(bs)
                return 0

            lax.fori_loop(0, CHUNK // LANES, hash_body, 0)

            def fire(b, _):
                pltpu.make_async_copy(tbl.at[idx_buf.at[b]],
                                      g_buf.at[pl.ds(b * 128, 128)], sem).start()
                return 0

            lax.fori_loop(0, NB, fire, 0)

            def drain(b, _):
                pltpu.make_async_copy(tbl.at[idx_buf.at[b]],
                                      g_buf.at[pl.ds(b * 128, 128)], sem).wait()
                return 0

            lax.fori_loop(0, NB, drain, 0)

            def accum_body(i, _, l=l):
                tx1 = xfx[pl.ds(i * LANES, LANES)]
                ty1 = xfy[pl.ds(i * LANES, LANES)]
                tz1 = xfz[pl.ds(i * LANES, LANES)]
                tx0 = 1.0 - tx1
                ty0 = 1.0 - ty1
                tz0 = 1.0 - tz1
                rv0 = iota + i * LANES
                acc = [jnp.zeros((LANES,), jnp.float32) for _ in range(4)]
                for c in range(8):
                    w = ((tx1 if (c >> 2) & 1 else tx0)
                         * (ty1 if (c >> 1) & 1 else ty0)
                         * (tz1 if c & 1 else tz0))
                    rv = rv0 + c * CHUNK
                    for f in range(4):
                        g = plsc.load_gather(g_buf, [rv, fvecs[f]])
                        acc[f] = acc[f] + g * w
                off = i * (LANES * 64) + l * 4
                for f in range(4):
                    plsc.store_scatter(feats_buf, [iota64 + (off + f)], acc[f])
                return 0

            lax.fori_loop(0, CHUNK // LANES, accum_body, 0)

        pltpu.sync_copy(feats_buf, out_hbm.at[pl.ds(start * 64, CHUNK * 64)])
        return 0

    lax.fori_loop(0, PW // CHUNK, chunk_body, 0)


@jax.jit
def _encode(x, y, z, tbl):
    fn = pl.kernel(
        _encode_body,
        out_type=jax.ShapeDtypeStruct((N_POINTS * 64,), jnp.float32),
        mesh=plsc.VectorSubcoreMesh(core_axis_name="c", subcore_axis_name="s"),
        compiler_params=pltpu.CompilerParams(
            needs_layout_passes=False, use_tc_tiling_on_sc=False
        ),
        scratch_types=[
            pltpu.VMEM((CHUNK,), jnp.float32),
            pltpu.VMEM((CHUNK,), jnp.float32),
            pltpu.VMEM((CHUNK,), jnp.float32),
            pltpu.VMEM((CHUNK,), jnp.float32),
            pltpu.VMEM((CHUNK,), jnp.float32),
            pltpu.VMEM((CHUNK,), jnp.float32),
            pltpu.VMEM((NB, 128), jnp.int32),
            pltpu.VMEM((NB * 128, 8), jnp.float32),
            pltpu.VMEM((CHUNK * 64,), jnp.float32),
            pltpu.SemaphoreType.DMA,
        ],
    )
    return fn(x, y, z, tbl)


BM = 2048


def _mlp_body(f_ref, w1_ref, b1_ref, w2_ref, b2_ref, w3_ref, b3_ref, o_ref):
    x = f_ref[...]
    h = jnp.dot(x, w1_ref[...], preferred_element_type=jnp.float32) + b1_ref[...]
    h = jnp.maximum(h, 0.0)
    h = jnp.dot(h, w2_ref[...], preferred_element_type=jnp.float32) + b2_ref[...]
    h = jnp.maximum(h, 0.0)
    t = jnp.dot(h, w3_ref[...], preferred_element_type=jnp.float32) + b3_ref[...]
    o_ref[...] = 1.0 / (1.0 + jnp.exp(-t))


@functools.partial(jax.jit, static_argnames=("n_out",))
def _mlp(feats, W1, b1, W2, b2, W3, b3, n_out):
    return pl.pallas_call(
        _mlp_body,
        grid=(N_POINTS // BM,),
        in_specs=[
            pl.BlockSpec((BM, 64), lambda i: (i, 0)),
            pl.BlockSpec((64, 64), lambda i: (0, 0)),
            pl.BlockSpec((1, 64), lambda i: (0, 0)),
            pl.BlockSpec((64, 64), lambda i: (0, 0)),
            pl.BlockSpec((1, 64), lambda i: (0, 0)),
            pl.BlockSpec((64, n_out), lambda i: (0, 0)),
            pl.BlockSpec((1, n_out), lambda i: (0, 0)),
        ],
        out_specs=pl.BlockSpec((BM, n_out), lambda i: (i, 0)),
        out_shape=jax.ShapeDtypeStruct((N_POINTS, n_out), jnp.float32),
    )(feats, W1, b1, W2, b2, W3, b3)


def kernel(ipos, tables, W1, b1, W2, b2, W3, b3):
    x = ipos[:, 0]
    y = ipos[:, 1]
    z = ipos[:, 2]
    tall = jnp.concatenate(
        [jnp.pad(t.T, ((0, 0), (0, pl_ - t.shape[0]))).reshape(-1)
         for t, pl_ in zip(tables, _PL)]
        + [jnp.zeros((4096,), jnp.float32)]
    )
    tbl = _relayout(tall).reshape(RP, 8)
    feats = _encode(x, y, z, tbl).reshape(N_POINTS, 64)
    n_out = W3.shape[1]
    return _mlp(feats, W1, b1[None, :], W2, b2[None, :], W3, b3[None, :], n_out)


# speedup vs baseline: 129.5992x; 1.0519x over previous
import functools
import math

import jax
import jax.numpy as jnp
import numpy as np
from jax import lax
from jax.experimental import pallas as pl
from jax.experimental.pallas import tpu as pltpu
from jax.experimental.pallas import tpu_sc as plsc

N_LEVELS = 16
N_FEATS = 4
LOG2_HASH = 18
BASE_RES = 16
FINEST_RES = 512
N_POINTS = 262144

_b = math.exp((math.log(FINEST_RES) - math.log(BASE_RES)) / (N_LEVELS - 1))
_RES = [int(math.floor(BASE_RES * (_b ** l))) for l in range(N_LEVELS)]
_SIZES = [min(r ** 3, 2 ** LOG2_HASH) for r in _RES]
_P2 = np.int32(np.uint32(2654435761).view(np.int32))
_P3 = np.int32(np.uint32(805459861).view(np.int32))

NW = 32
LANES = 16
CHUNK = 512
PW = N_POINTS // NW
CPB = CHUNK // 128
NB = 8 * CPB


def _is_pow2(m):
    return (m & (m - 1)) == 0


def _umod(h, m):
    hf = h.astype(jnp.float32)
    hf = jnp.where(h < 0, hf + jnp.float32(4294967296.0), hf)
    q = (hf * jnp.float32(1.0 / m)).astype(jnp.int32)
    r = h - q * jnp.int32(m)
    r = jnp.where(r < 0, r + jnp.int32(m), r)
    r = jnp.where(r >= jnp.int32(m), r - jnp.int32(m), r)
    return r


_BASES = [0]
for _s in _SIZES:
    _BASES.append(_BASES[-1] + _s)
TOTAL_ROWS = _BASES[N_LEVELS]

_CH = [-(-(-(-s // -NW)) // -LANES) * LANES for s in _SIZES]
_CH = [((s + NW - 1) // NW + LANES - 1) // LANES * LANES for s in _SIZES]
_BP = [0]
for _c in _CH:
    _BP.append(_BP[-1] + NW * _c)
RP = _BP[N_LEVELS]
_PL = [(s + 7) // 8 * 8 for s in _SIZES]
_SEG = [0]
for _p in _PL:
    _SEG.append(_SEG[-1] + 4 * _p)


def _relayout_body(src_hbm, big_hbm, p0, p1, p2, p3, rowbuf, sem):
    wid = lax.axis_index("s") * 2 + lax.axis_index("c")
    iota = lax.iota(jnp.int32, LANES)
    pat = iota * 8
    planes = (p0, p1, p2, p3)
    for l in range(N_LEVELS):
        ch = _CH[l]
        seg = _SEG[l]
        pln = _PL[l]
        r0 = wid * ch
        for f in range(4):
            pltpu.sync_copy(src_hbm.at[pl.ds(seg + f * pln + r0, ch)],
                            planes[f].at[pl.ds(0, ch)])

        def grp(g, _, ch=ch):
            for f in range(4):
                v = planes[f][pl.ds(g * LANES, LANES)]
                plsc.store_scatter(rowbuf, [pat + (g * 128 + f)], v)
            return 0

        lax.fori_loop(0, ch // LANES, grp, 0)
        pltpu.sync_copy(rowbuf.at[pl.ds(0, ch * 8)],
                        big_hbm.at[pl.ds((_BP[l] + r0) * 8, ch * 8)])


@jax.jit
def _relayout(tall):
    fn = pl.kernel(
        _relayout_body,
        out_type=jax.ShapeDtypeStruct((RP * 8,), jnp.float32),
        mesh=plsc.VectorSubcoreMesh(core_axis_name="c", subcore_axis_name="s"),
        compiler_params=pltpu.CompilerParams(
            needs_layout_passes=False, use_tc_tiling_on_sc=False
        ),
        scratch_types=[
            pltpu.VMEM((max(_CH),), jnp.float32),
            pltpu.VMEM((max(_CH),), jnp.float32),
            pltpu.VMEM((max(_CH),), jnp.float32),
            pltpu.VMEM((max(_CH),), jnp.float32),
            pltpu.VMEM((max(_CH) * 8,), jnp.float32),
            pltpu.SemaphoreType.DMA,
        ],
    )
    return fn(tall)


def _encode_body(x_hbm, y_hbm, z_hbm, tbl, out_hbm,
                 xb, yb, zb, xf0, xf1, idx0, idx1, g0, g1, feats_buf, sem):

    wid = lax.axis_index("s") * 2 + lax.axis_index("c")
    base = wid * PW
    iota = lax.iota(jnp.int32, LANES)
    iota64 = iota * 64
    fvecs = [jnp.full((LANES,), f, jnp.int32) for f in range(4)]
    xfs = (xf0, xf1)
    idxs = (idx0, idx1)
    gs = (g0, g1)

    def hash_level(l, idx_buf, xf):
        r = _RES[l]
        m = _SIZES[l]
        bs = _BP[l]

        def hash_body(i, _):
            xs = xb[pl.ds(i * LANES, LANES)] * jnp.float32(r)
            ys = yb[pl.ds(i * LANES, LANES)] * jnp.float32(r)
            zs = zb[pl.ds(i * LANES, LANES)] * jnp.float32(r)
            xi = xs.astype(jnp.int32)
            yi = ys.astype(jnp.int32)
            zi = zs.astype(jnp.int32)
            xf[0, pl.ds(i * LANES, LANES)] = xs - xi.astype(jnp.float32)
            xf[1, pl.ds(i * LANES, LANES)] = ys - yi.astype(jnp.float32)
            xf[2, pl.ds(i * LANES, LANES)] = zs - zi.astype(jnp.float32)
            hx0, hx1 = xi, xi + 1
            hy0 = yi * _P2
            hy1 = hy0 + _P2
            hz0 = zi * _P3
            hz1 = hz0 + _P3
            r0 = i * LANES - (i // 8) * 128
            for c in range(8):
                h = (hx1 if (c >> 2) & 1 else hx0) ^ \
                    (hy1 if (c >> 1) & 1 else hy0) ^ \
                    (hz1 if c & 1 else hz0)
                if _is_pow2(m):
                    idx = h & jnp.int32(m - 1)
                else:
                    idx = _umod(h, m)
                idx_buf[c * CPB + i // 8, pl.ds(r0, LANES)] = idx + jnp.int32(bs)
            return 0

        lax.fori_loop(0, CHUNK // LANES, hash_body, 0)

    def fire_level(idx_buf, g_buf):
        def fire(b, _):
            pltpu.make_async_copy(tbl.at[idx_buf.at[b]],
                                  g_buf.at[pl.ds(b * 128, 128)], sem).start()
            return 0

        lax.fori_loop(0, NB, fire, 0)

    def drain_level(idx_buf, g_buf):
        def drain(b, _):
            pltpu.make_async_copy(tbl.at[idx_buf.at[b]],
                                  g_buf.at[pl.ds(b * 128, 128)], sem).wait()
            return 0

        lax.fori_loop(0, NB, drain, 0)

    def accum_level(l, g_buf, xf):
        def accum_body(i, _):
            tx1 = xf[0, pl.ds(i * LANES, LANES)]
            ty1 = xf[1, pl.ds(i * LANES, LANES)]
            tz1 = xf[2, pl.ds(i * LANES, LANES)]
            tx0 = 1.0 - tx1
            ty0 = 1.0 - ty1
            tz0 = 1.0 - tz1
            rv0 = iota + i * LANES
            acc = [jnp.zeros((LANES,), jnp.float32) for _ in range(4)]
            for c in range(8):
                w = ((tx1 if (c >> 2) & 1 else tx0)
                     * (ty1 if (c >> 1) & 1 else ty0)
                     * (tz1 if c & 1 else tz0))
                rv = rv0 + c * CHUNK
                for f in range(4):
                    g = plsc.load_gather(g_buf, [rv, fvecs[f]])
                    acc[f] = acc[f] + g * w
            off = i * (LANES * 64) + l * 4
            for f in range(4):
                plsc.store_scatter(feats_buf, [iota64 + (off + f)], acc[f])
            return 0

        lax.fori_loop(0, CHUNK // LANES, accum_body, 0)

    def chunk_body(k, _):
        start = base + k * CHUNK
        pltpu.sync_copy(x_hbm.at[pl.ds(start, CHUNK)], xb)
        pltpu.sync_copy(y_hbm.at[pl.ds(start, CHUNK)], yb)
        pltpu.sync_copy(z_hbm.at[pl.ds(start, CHUNK)], zb)

        hash_level(0, idxs[0], xfs[0])
        for l in range(N_LEVELS):
            p = l % 2
            fire_level(idxs[p], gs[p])
            if l + 1 < N_LEVELS:
                hash_level(l + 1, idxs[1 - p], xfs[1 - p])
            drain_level(idxs[p], gs[p])
            accum_level(l, gs[p], xfs[p])

        pltpu.sync_copy(feats_buf, out_hbm.at[pl.ds(start * 64, CHUNK * 64)])
        return 0

    lax.fori_loop(0, PW // CHUNK, chunk_body, 0)


@jax.jit
def _encode(x, y, z, tbl):
    fn = pl.kernel(
        _encode_body,
        out_type=jax.ShapeDtypeStruct((N_POINTS * 64,), jnp.float32),
        mesh=plsc.VectorSubcoreMesh(core_axis_name="c", subcore_axis_name="s"),
        compiler_params=pltpu.CompilerParams(
            needs_layout_passes=False, use_tc_tiling_on_sc=False
        ),
        scratch_types=[
            pltpu.VMEM((CHUNK,), jnp.float32),
            pltpu.VMEM((CHUNK,), jnp.float32),
            pltpu.VMEM((CHUNK,), jnp.float32),
            pltpu.VMEM((3, CHUNK), jnp.float32),
            pltpu.VMEM((3, CHUNK), jnp.float32),
            pltpu.VMEM((NB, 128), jnp.int32),
            pltpu.VMEM((NB, 128), jnp.int32),
            pltpu.VMEM((NB * 128, 8), jnp.float32),
            pltpu.VMEM((NB * 128, 8), jnp.float32),
            pltpu.VMEM((CHUNK * 64,), jnp.float32),
            pltpu.SemaphoreType.DMA,
        ],
    )
    return fn(x, y, z, tbl)


BM = 2048


def _mlp_body(f_ref, w1_ref, b1_ref, w2_ref, b2_ref, w3_ref, b3_ref, o_ref):
    x = f_ref[...]
    h = jnp.dot(x, w1_ref[...], preferred_element_type=jnp.float32) + b1_ref[...]
    h = jnp.maximum(h, 0.0)
    h = jnp.dot(h, w2_ref[...], preferred_element_type=jnp.float32) + b2_ref[...]
    h = jnp.maximum(h, 0.0)
    t = jnp.dot(h, w3_ref[...], preferred_element_type=jnp.float32) + b3_ref[...]
    o_ref[...] = 1.0 / (1.0 + jnp.exp(-t))


@functools.partial(jax.jit, static_argnames=("n_out",))
def _mlp(feats, W1, b1, W2, b2, W3, b3, n_out):
    return pl.pallas_call(
        _mlp_body,
        grid=(N_POINTS // BM,),
        in_specs=[
            pl.BlockSpec((BM, 64), lambda i: (i, 0)),
            pl.BlockSpec((64, 64), lambda i: (0, 0)),
            pl.BlockSpec((1, 64), lambda i: (0, 0)),
            pl.BlockSpec((64, 64), lambda i: (0, 0)),
            pl.BlockSpec((1, 64), lambda i: (0, 0)),
            pl.BlockSpec((64, n_out), lambda i: (0, 0)),
            pl.BlockSpec((1, n_out), lambda i: (0, 0)),
        ],
        out_specs=pl.BlockSpec((BM, n_out), lambda i: (i, 0)),
        out_shape=jax.ShapeDtypeStruct((N_POINTS, n_out), jnp.float32),
    )(feats, W1, b1, W2, b2, W3, b3)


def kernel(ipos, tables, W1, b1, W2, b2, W3, b3):
    x = ipos[:, 0]
    y = ipos[:, 1]
    z = ipos[:, 2]
    tall = jnp.concatenate(
        [jnp.pad(t.T, ((0, 0), (0, pl_ - t.shape[0]))).reshape(-1)
         for t, pl_ in zip(tables, _PL)]
        + [jnp.zeros((4096,), jnp.float32)]
    )
    tbl = _relayout(tall).reshape(RP, 8)
    feats = _encode(x, y, z, tbl).reshape(N_POINTS, 64)
    n_out = W3.shape[1]
    return _mlp(feats, W1, b1[None, :], W2, b2[None, :], W3, b3[None, :], n_out)
